# baseline (speedup 1.0000x reference)
import functools

import jax
import jax.numpy as jnp
from jax import lax
from jax.experimental import pallas as pl

_N_E = 1024
_E_DIM = 256
_BETA = 0.25
_B = 8
_HW = 576
_G = 3
_TOK = (_B * _HW) // _G


def _vq_body(zf_ref, cb_ref, out_ref, counts_ref, loss_ref):
    b = pl.program_id(0)
    zfb = zf_ref[...]
    cb = cb_ref[...]

    mm2 = lax.dot_general(-2.0 * zfb, cb, (((1,), (1,)), ((), ())))
    zsq = jnp.sum(zfb * zfb, axis=1, keepdims=True)
    esq = jnp.sum(cb * cb, axis=1)
    d = (zsq + esq) + mm2

    dmin = jnp.min(d, axis=1, keepdims=True)
    ids = lax.broadcasted_iota(jnp.int32, (_TOK, _N_E), 1)
    idx = jnp.min(jnp.where(d == dmin, ids, _N_E), axis=1, keepdims=True)

    onehot = (ids == idx).astype(jnp.float32)
    zq = lax.dot_general(onehot, cb, (((1,), (0,)), ((), ())))
    out_ref[...] = zq

    ones_row = jnp.full((8, _TOK), 1.0, dtype=jnp.float32)
    cpart = lax.dot_general(ones_row, onehot,
                            (((1,), (0,)), ((), ())))[0]
    lpart = jnp.sum(dmin, axis=0, keepdims=True)

    @pl.when(b == 0)
    def _init():
        counts_ref[...] = cpart
        loss_ref[...] = lpart

    @pl.when(b > 0)
    def _acc():
        counts_ref[...] = counts_ref[...] + cpart
        loss_ref[...] = loss_ref[...] + lpart


@functools.partial(jax.jit, static_argnames=("interpret",))
def kernel(z, codebook, interpret=False):
    zf = jnp.transpose(z, (0, 2, 3, 1)).reshape(_G, _TOK, _E_DIM)
    out3, counts, loss_sum = pl.pallas_call(
        _vq_body,
        grid=(_G,),
        in_specs=[
            pl.BlockSpec((None, _TOK, _E_DIM), lambda b: (b, 0, 0)),
            pl.BlockSpec((_N_E, _E_DIM), lambda b: (0, 0)),
        ],
        out_specs=[
            pl.BlockSpec((None, _TOK, _E_DIM), lambda b: (b, 0, 0)),
            pl.BlockSpec((_N_E,), lambda b: (0,)),
            pl.BlockSpec((1, 1), lambda b: (0, 0)),
        ],
        out_shape=[
            jax.ShapeDtypeStruct((_G, _TOK, _E_DIM), jnp.float32),
            jax.ShapeDtypeStruct((_N_E,), jnp.float32),
            jax.ShapeDtypeStruct((1, 1), jnp.float32),
        ],
        interpret=interpret,
    )(zf, codebook)

    z_q_out = jnp.transpose(out3.reshape(_B, 24, 24, _E_DIM), (0, 3, 1, 2))
    n = _B * _HW * _E_DIM
    l_mean = loss_sum[0, 0] / n
    loss = _BETA * l_mean + l_mean
    return (z_q_out, loss, counts)


# speedup vs baseline: 1.0050x; 1.0050x over previous
import jax
import jax.numpy as jnp
from jax import lax
from jax.experimental import pallas as pl

_N_E = 1024
_E_DIM = 256
_BETA = 0.25
_B = 8
_HW = 576
_G = 2
_TOK = (_B * _HW) // _G


def _vq_body(zf_ref, cb_ref, out_ref, counts_ref, loss_ref):
    b = pl.program_id(0)
    zfb = zf_ref[...]
    cb = cb_ref[...]

    mm2 = lax.dot_general(-2.0 * zfb, cb, (((1,), (1,)), ((), ())))
    zsq = jnp.sum(zfb * zfb, axis=1, keepdims=True)
    esq = jnp.sum(cb * cb, axis=1)
    d = (zsq + esq) + mm2

    dmin = jnp.min(d, axis=1, keepdims=True)
    ids = lax.broadcasted_iota(jnp.int32, (_TOK, _N_E), 1)
    idx = jnp.min(jnp.where(d == dmin, ids, _N_E), axis=1, keepdims=True)

    onehot = (ids == idx).astype(jnp.float32)
    zq = lax.dot_general(onehot, cb, (((1,), (0,)), ((), ())))
    out_ref[...] = zq

    ones_row = jnp.full((8, _TOK), 1.0, dtype=jnp.float32)
    cpart = lax.dot_general(ones_row, onehot,
                            (((1,), (0,)), ((), ())))[0]
    lpart = jnp.sum(dmin, axis=0, keepdims=True)

    @pl.when(b == 0)
    def _init():
        counts_ref[...] = cpart
        loss_ref[...] = lpart

    @pl.when(b > 0)
    def _acc():
        counts_ref[...] = counts_ref[...] + cpart
        loss_ref[...] = loss_ref[...] + lpart


@jax.jit
def kernel(z, codebook):
    zf = jnp.transpose(z, (0, 2, 3, 1)).reshape(_G, _TOK, _E_DIM)
    out3, counts, loss_sum = pl.pallas_call(
        _vq_body,
        grid=(_G,),
        in_specs=[
            pl.BlockSpec((None, _TOK, _E_DIM), lambda b: (b, 0, 0)),
            pl.BlockSpec((_N_E, _E_DIM), lambda b: (0, 0)),
        ],
        out_specs=[
            pl.BlockSpec((None, _TOK, _E_DIM), lambda b: (b, 0, 0)),
            pl.BlockSpec((_N_E,), lambda b: (0,)),
            pl.BlockSpec((1, 1), lambda b: (0, 0)),
        ],
        out_shape=[
            jax.ShapeDtypeStruct((_G, _TOK, _E_DIM), jnp.float32),
            jax.ShapeDtypeStruct((_N_E,), jnp.float32),
            jax.ShapeDtypeStruct((1, 1), jnp.float32),
        ],
    )(zf, codebook)

    z_q_out = jnp.transpose(out3.reshape(_B, 24, 24, _E_DIM), (0, 3, 1, 2))
    n = _B * _HW * _E_DIM
    l_mean = loss_sum[0, 0] / n
    loss = _BETA * l_mean + l_mean
    return (z_q_out, loss, counts)
